# baseline (speedup 1.0000x reference)
import functools

import jax
import jax.numpy as jnp
from jax import lax
from jax.experimental import pallas as pl
from jax.experimental.pallas import tpu as pltpu
from jax.experimental.pallas import tpu_sc as plsc

NUM_ROWS = 1000000
DIM = 64
BATCH = 4096
SEQ = 200
NC, NS = 2, 16
NW = NC * NS
ROWS_PER_W = BATCH // NW
R = 4
NG = ROWS_PER_W // R

_mesh = plsc.VectorSubcoreMesh(core_axis_name="c", subcore_axis_name="s")


@functools.partial(
    pl.kernel,
    mesh=_mesh,
    out_type=jax.ShapeDtypeStruct((BATCH, SEQ, DIM), jnp.float32),
    scratch_types=[
        pltpu.VMEM((ROWS_PER_W * SEQ,), jnp.int32),
        pltpu.VMEM((2, R * SEQ, DIM), jnp.float32),
        pltpu.SemaphoreType.DMA,
        pltpu.SemaphoreType.DMA,
    ],
    compiler_params=pltpu.CompilerParams(use_tc_tiling_on_sc=False),
)
def _gather_kernel(idx_hbm, table_hbm, out_hbm, idx_v, rows_v, gsem, ssem):
    wid = lax.axis_index("s") * NC + lax.axis_index("c")
    base = wid * ROWS_PER_W
    pltpu.sync_copy(idx_hbm.at[pl.ds(base * SEQ, ROWS_PER_W * SEQ)], idx_v)

    def gather(g, buf):
        pltpu.async_copy(
            table_hbm.at[idx_v.at[pl.ds(g * R * SEQ, R * SEQ)]],
            rows_v.at[buf],
            gsem,
        )

    def store(g, buf):
        for r in range(R):
            pltpu.async_copy(
                rows_v.at[buf, pl.ds(r * SEQ, SEQ)],
                out_hbm.at[base + g * R + r],
                ssem,
            )

    def wait_gather(buf):
        pltpu.make_async_copy(
            table_hbm.at[pl.ds(0, R * SEQ)], rows_v.at[buf], gsem
        ).wait()

    def wait_store(buf):
        for r in range(R):
            pltpu.make_async_copy(
                rows_v.at[buf, pl.ds(r * SEQ, SEQ)], out_hbm.at[base], ssem
            ).wait()

    def step(g, b):
        wait_gather(b)
        store(g, b)
        wait_store(1 - b)
        gather(g + 1, 1 - b)

    gather(0, 0)
    gather(1, 1)
    wait_gather(0)
    store(0, 0)

    def body(i, carry):
        step(2 * i + 1, 1)
        step(2 * i + 2, 0)
        return carry

    lax.fori_loop(0, (NG - 2) // 2, body, 0)

    wait_gather(1)
    store(NG - 1, 1)
    wait_store(0)
    wait_store(1)


def kernel(input_ids, weight):
    flat_ids = input_ids.astype(jnp.int32).reshape(-1)
    return _gather_kernel(flat_ids, weight)
